# baseline (speedup 1.0000x reference)
import jax
import jax.numpy as jnp
from jax import lax
from jax.experimental import pallas as pl
from jax.experimental.pallas import tpu as pltpu
from jax.experimental.pallas import tpu_sc as plsc

_L = 16
_NW = 32


def _gather_body(ids_hbm, tok_hbm, out_hbm, idx_v, rows_v, sem, osem):
    b, seq_len = ids_hbm.shape
    h = tok_hbm.shape[1]
    t_per = (b * seq_len) // _NW
    n_grp = t_per // h
    chunks_per_seq = seq_len // t_per

    wid = lax.axis_index("s") * 2 + lax.axis_index("c")
    bi = wid // chunks_per_seq
    s0 = lax.rem(wid, chunks_per_seq) * t_per
    base = wid * t_per

    pltpu.sync_copy(ids_hbm.at[bi, pl.ds(s0, t_per)], idx_v)

    copies = []
    for k in range(n_grp):
        copies.append(pltpu.async_copy(
            tok_hbm.at[idx_v.at[pl.ds(k * h, h)]],
            rows_v.at[pl.ds(k * h, h)], sem))
    out_cps = []
    for k in range(n_grp):
        copies[k].wait()
        out_cps.append(pltpu.async_copy(
            rows_v.at[pl.ds(k * h, h)],
            out_hbm.at[pl.ds(base + k * h, h)], osem))
    for c in out_cps:
        c.wait()


def _ln_body(tok_ref, ps_ref, gam_ref, bet_ref, acc_ref, out_ref):
    del acc_ref
    e = tok_ref[...] + ps_ref[...]
    mean = jnp.mean(e, axis=-1, keepdims=True)
    c = e - mean
    var = jnp.mean(c * c, axis=-1, keepdims=True)
    r = lax.rsqrt(var + jnp.float32(1e-5))
    out_ref[...] = c * r * gam_ref[...] + bet_ref[...]


def kernel(input_ids, token_type_ids, tok_table, pos_table, seg_table,
           gamma, beta):
    b, s = input_ids.shape
    v, h = tok_table.shape
    n = b * s
    bh = b // 2
    nh = bh * s
    t_per = nh // _NW
    ids = input_ids.astype(jnp.int32)

    mesh = plsc.VectorSubcoreMesh(core_axis_name="c", subcore_axis_name="s")
    sc_gather = pl.kernel(
        _gather_body,
        out_type=jax.ShapeDtypeStruct((nh, h), jnp.float32),
        mesh=mesh,
        scratch_types=[
            pltpu.VMEM((t_per,), jnp.int32),
            pltpu.VMEM((t_per, h), jnp.float32),
            pltpu.SemaphoreType.DMA,
            pltpu.SemaphoreType.DMA,
        ],
    )

    gam2 = gamma.reshape(1, h)
    bet2 = beta.reshape(1, h)
    out = jnp.zeros((n, h), jnp.float32)
    for half in range(2):
        ids_h = ids[half * bh:(half + 1) * bh]
        tt_h = token_type_ids[half * bh:(half + 1) * bh]
        tok_rows = sc_gather(ids_h, tok_table)
        posseg = (pos_table[None, :, :]
                  + jnp.where((tt_h == 0)[..., None],
                              seg_table[0], seg_table[1])).reshape(nh, h)
        out = pl.pallas_call(
            _ln_body,
            grid=(1,),
            in_specs=[
                pl.BlockSpec((nh, h), lambda i: (0, 0)),
                pl.BlockSpec((nh, h), lambda i: (0, 0)),
                pl.BlockSpec((1, h), lambda i: (0, 0)),
                pl.BlockSpec((1, h), lambda i: (0, 0)),
                pl.BlockSpec((8, h), lambda i: (0, 0)),
            ],
            out_specs=pl.BlockSpec((nh, h),
                                   lambda i, _half=half: (_half, 0)),
            out_shape=jax.ShapeDtypeStruct((n, h), jnp.float32),
            input_output_aliases={4: 0},
        )(tok_rows, posseg, gam2, bet2, out)
    return out.reshape(b, s, h)


# speedup vs baseline: 1.1634x; 1.1634x over previous
import jax
import jax.numpy as jnp
from jax import lax
from jax.experimental import pallas as pl
from jax.experimental.pallas import tpu as pltpu
from jax.experimental.pallas import tpu_sc as plsc

_L = 16
_NW = 32


def _gather_body(ids_hbm, tok_hbm, out_hbm, idx_v, rows_v, osem, *gsems):
    b, seq_len = ids_hbm.shape
    h = tok_hbm.shape[1]
    t_per = (b * seq_len) // _NW
    n_grp = t_per // h
    chunks_per_seq = seq_len // t_per

    wid = lax.axis_index("s") * 2 + lax.axis_index("c")
    bi = wid // chunks_per_seq
    s0 = lax.rem(wid, chunks_per_seq) * t_per
    base = wid * t_per

    pltpu.sync_copy(ids_hbm.at[bi, pl.ds(s0, t_per)], idx_v)

    gw = t_per // len(gsems)
    n_grp = len(gsems)
    copies = []
    for k in range(n_grp):
        copies.append(pltpu.async_copy(
            tok_hbm.at[idx_v.at[pl.ds(k * gw, gw)]],
            rows_v.at[pl.ds(k * gw, gw)], gsems[k]))
    out_cps = []
    for k in range(n_grp):
        copies[k].wait()
        out_cps.append(pltpu.async_copy(
            rows_v.at[pl.ds(k * gw, gw)],
            out_hbm.at[pl.ds(base + k * gw, gw)], osem))
    for c in out_cps:
        c.wait()


def _ln_body(tok_ref, ps_ref, gam_ref, bet_ref, out_ref):
    e = tok_ref[...] + ps_ref[...]
    mean = jnp.mean(e, axis=-1, keepdims=True)
    c = e - mean
    var = jnp.mean(c * c, axis=-1, keepdims=True)
    r = lax.rsqrt(var + jnp.float32(1e-5))
    out_ref[...] = c * r * gam_ref[...] + bet_ref[...]


def kernel(input_ids, token_type_ids, tok_table, pos_table, seg_table,
           gamma, beta):
    b, s = input_ids.shape
    v, h = tok_table.shape
    n = b * s
    t_per = n // _NW
    ids = input_ids.astype(jnp.int32)

    mesh = plsc.VectorSubcoreMesh(core_axis_name="c", subcore_axis_name="s")
    tok_rows = pl.kernel(
        _gather_body,
        out_type=jax.ShapeDtypeStruct((n, h), jnp.float32),
        mesh=mesh,
        scratch_types=[
            pltpu.VMEM((t_per,), jnp.int32),
            pltpu.VMEM((t_per, h), jnp.float32),
            pltpu.SemaphoreType.DMA,
        ] + [pltpu.SemaphoreType.DMA] * 4,
    )(ids, tok_table)

    posseg = (pos_table[None, :, :]
              + jnp.where((token_type_ids == 0)[..., None],
                          seg_table[0], seg_table[1])).reshape(n, h)

    br = 4096
    out = pl.pallas_call(
        _ln_body,
        grid=(n // br,),
        in_specs=[
            pl.BlockSpec((br, h), lambda i: (i, 0)),
            pl.BlockSpec((br, h), lambda i: (i, 0)),
            pl.BlockSpec((1, h), lambda i: (0, 0)),
            pl.BlockSpec((1, h), lambda i: (0, 0)),
        ],
        out_specs=pl.BlockSpec((br, h), lambda i: (i, 0)),
        out_shape=jax.ShapeDtypeStruct((n, h), jnp.float32),
    )(tok_rows, posseg, gamma.reshape(1, h), beta.reshape(1, h))
    return out.reshape(b, s, h)
